# baseline (speedup 1.0000x reference)
import functools

import jax
import jax.numpy as jnp
from jax import lax
from jax.experimental import pallas as pl
from jax.experimental.pallas import tpu as pltpu
from jax.experimental.pallas import tpu_sc as plsc

N_ROWS = 10000
NRP = 10240
D = 256
DP = 128
E = 160000
EPS = 1e-8

NC, NS, L = 2, 16, 16
DPH = DP // NC
BLK = 64
NSPLIT = 2
TPW = 10240
E_PAD = NS * TPW
NBLK = TPW // BLK
RPT = NRP // NS


def _norm_body(x_ref, o_ref):
    x = x_ref[...]
    n = jnp.sqrt(jnp.sum(x * x, axis=1, keepdims=True))
    xn = x / jnp.maximum(n, EPS)
    a = jax.lax.bitcast_convert_type(
        xn[:, :DP].astype(jnp.bfloat16), jnp.uint16).astype(jnp.uint32)
    b = jax.lax.bitcast_convert_type(
        xn[:, DP:].astype(jnp.bfloat16), jnp.uint16).astype(jnp.uint32)
    o_ref[...] = jax.lax.bitcast_convert_type(a | (b << 16), jnp.int32)


def _normalize(x):
    return pl.pallas_call(
        _norm_body,
        grid=(5,),
        in_specs=[pl.BlockSpec((2000, D), lambda i: (i, 0))],
        out_specs=pl.BlockSpec((2000, DP), lambda i: (i, 0)),
        out_shape=jax.ShapeDtypeStruct((N_ROWS, DP), jnp.int32),
    )(x)


def _comb_body(p_ref, o_ref):
    o_ref[...] = p_ref[0] + p_ref[1]


def _combine(parts):
    return pl.pallas_call(
        _comb_body,
        grid=(8,),
        in_specs=[pl.BlockSpec((NC, 160, 128), lambda i: (0, i, 0))],
        out_specs=pl.BlockSpec((160, 128), lambda i: (i, 0)),
        out_shape=jax.ShapeDtypeStruct((E_PAD // 128, 128), jnp.float32),
    )(parts)


@functools.partial(
    pl.kernel,
    out_type=jax.ShapeDtypeStruct((NC, E_PAD), jnp.float32),
    mesh=plsc.VectorSubcoreMesh(core_axis_name="c", subcore_axis_name="s"),
    compiler_params=pltpu.CompilerParams(
        use_tc_tiling_on_sc=False, needs_layout_passes=False),
    scratch_types=[
        pltpu.VMEM((NBLK * NSPLIT, BLK // NSPLIT), jnp.int32),
        pltpu.VMEM((NBLK * NSPLIT, BLK // NSPLIT), jnp.int32),
        pltpu.VMEM_SHARED((NRP, DPH), jnp.int32),
        pltpu.VMEM_SHARED((NRP, DPH), jnp.int32),
        pltpu.VMEM((BLK, DPH), jnp.int32),
        pltpu.VMEM((BLK, DPH), jnp.int32),
        pltpu.VMEM((BLK, DPH), jnp.int32),
        pltpu.VMEM((BLK, DPH), jnp.int32),
        pltpu.VMEM((TPW,), jnp.float32),
        pltpu.SemaphoreType.DMA,
        pltpu.SemaphoreType.DMA,
        pltpu.SemaphoreType.DMA,
        pltpu.SemaphoreType.DMA,
    ],
)
def _sc_dot(u_hbm, v_hbm, src_hbm, dst_hbm, out_hbm,
            idx_u, idx_v, u_sp, v_sp, u0, u1, v0, v1, out_all,
            su0, su1, sv0, sv1):
    cid = lax.axis_index("c")
    sid = lax.axis_index("s")
    lanes = lax.iota(jnp.int32, L)
    zero = jnp.zeros((L,), jnp.float32)
    ubufs, vbufs = (u0, u1), (v0, v1)
    usems, vsems = (su0, su1), (sv0, sv1)

    rs = pl.ds(sid * RPT, RPT)
    pltpu.sync_copy(u_hbm.at[cid, rs], u_sp.at[rs])
    pltpu.sync_copy(v_hbm.at[cid, rs], v_sp.at[rs])
    pltpu.sync_copy(src_hbm.at[sid], idx_u)
    pltpu.sync_copy(dst_hbm.at[sid], idx_v)
    plsc.subcore_barrier()

    def fire(g, b):
        for s in range(NSPLIT):
            sl = pl.ds(s * (BLK // NSPLIT), BLK // NSPLIT)
            pltpu.async_copy(
                u_sp.at[idx_u.at[g * NSPLIT + s]], ubufs[b].at[sl], usems[b])
            pltpu.async_copy(
                v_sp.at[idx_v.at[g * NSPLIT + s]], vbufs[b].at[sl], vsems[b])

    def wait(g, b):
        for s in range(NSPLIT):
            sl = pl.ds(s * (BLK // NSPLIT), BLK // NSPLIT)
            pltpu.make_async_copy(
                u_sp.at[idx_u.at[g * NSPLIT + s]], ubufs[b].at[sl],
                usems[b]).wait()
            pltpu.make_async_copy(
                v_sp.at[idx_v.at[g * NSPLIT + s]], vbufs[b].at[sl],
                vsems[b]).wait()

    fire(0, 0)

    def pair_body(i, carry):
        for b in range(2):
            g = 2 * i + b

            @pl.when(g + 1 < NBLK)
            def _():
                fire(g + 1, 1 - b)

            wait(g, b)
            ur, vr = ubufs[b], vbufs[b]

            def d_body(j, accs):
                res = list(accs)
                for t in range(4):
                    d = j * 4 + t
                    dcol = (jnp.full((L,), d, jnp.int32) + lanes) & (DPH - 1)
                    for grp in range(BLK // L):
                        rows = lanes + grp * L
                        uc = plsc.load_gather(ur, [rows, dcol])
                        vc = plsc.load_gather(vr, [rows, dcol])
                        prod = (plsc.bitcast(uc, jnp.bfloat16)
                                * plsc.bitcast(vc, jnp.bfloat16))
                        pa, pb = plsc.unpack(
                            prod, format=plsc.PackFormat.INTERLEAVED)
                        k = grp * 2 + (t & 1)
                        res[k] = res[k] + pa + pb
                return tuple(res)

            accs = lax.fori_loop(0, DPH // 4, d_body, (zero,) * (2 * BLK // L))
            for grp in range(BLK // L):
                acc = accs[grp * 2] + accs[grp * 2 + 1]
                out_all[pl.ds(g * BLK + grp * L, L)] = acc
        return carry

    lax.fori_loop(0, NBLK // 2, pair_body, 0)
    pltpu.sync_copy(out_all, out_hbm.at[cid, pl.ds(sid * TPW, TPW)])


def kernel(x_user, x_venue, edge_label_index):
    u_pk = _normalize(x_user)
    v_pk = _normalize(x_venue)
    rpad = ((0, NRP - N_ROWS), (0, 0), (0, 0))
    u3 = jnp.pad(u_pk.reshape(N_ROWS, NC, DPH).transpose(1, 0, 2),
                 ((0, 0),) + rpad[:2])
    v3 = jnp.pad(v_pk.reshape(N_ROWS, NC, DPH).transpose(1, 0, 2),
                 ((0, 0),) + rpad[:2])
    eli = edge_label_index.astype(jnp.int32)
    pad = jnp.zeros((E_PAD - E,), jnp.int32)
    src = jnp.concatenate([eli[0], pad]).reshape(NS, NBLK * NSPLIT,
                                                 BLK // NSPLIT)
    dst = jnp.concatenate([eli[1], pad]).reshape(NS, NBLK * NSPLIT,
                                                 BLK // NSPLIT)
    parts = _sc_dot(u3, v3, src, dst)
    out = _combine(parts.reshape(NC, E_PAD // 128, 128))
    return out.reshape(E_PAD)[:E]


# speedup vs baseline: 1.0704x; 1.0704x over previous
import functools

import jax
import jax.numpy as jnp
from jax import lax
from jax.experimental import pallas as pl
from jax.experimental.pallas import tpu as pltpu
from jax.experimental.pallas import tpu_sc as plsc

N_ROWS = 10000
NRP = 10240
D = 256
DP = 128
E = 160000
EPS = 1e-8

NC, NS, L = 2, 16, 16
DPH = DP // NC
BLK = 64
NSPLIT = 1
TPW = 10240
E_PAD = NS * TPW
NBLK = TPW // BLK
RPT = NRP // NS


def _norm_body(x_ref, o_ref):
    x = x_ref[...]
    n = jnp.sqrt(jnp.sum(x * x, axis=1, keepdims=True))
    xn = x / jnp.maximum(n, EPS)
    a = jax.lax.bitcast_convert_type(
        xn[:, :DP].astype(jnp.bfloat16), jnp.uint16).astype(jnp.uint32)
    b = jax.lax.bitcast_convert_type(
        xn[:, DP:].astype(jnp.bfloat16), jnp.uint16).astype(jnp.uint32)
    o_ref[...] = jax.lax.bitcast_convert_type(a | (b << 16), jnp.int32)


def _normalize(x):
    return pl.pallas_call(
        _norm_body,
        grid=(5,),
        in_specs=[pl.BlockSpec((2000, D), lambda i: (i, 0))],
        out_specs=pl.BlockSpec((2000, DP), lambda i: (i, 0)),
        out_shape=jax.ShapeDtypeStruct((N_ROWS, DP), jnp.int32),
    )(x)


def _comb_body(p_ref, o_ref):
    o_ref[...] = p_ref[0] + p_ref[1]


def _combine(parts):
    return pl.pallas_call(
        _comb_body,
        grid=(8,),
        in_specs=[pl.BlockSpec((NC, 160, 128), lambda i: (0, i, 0))],
        out_specs=pl.BlockSpec((160, 128), lambda i: (i, 0)),
        out_shape=jax.ShapeDtypeStruct((E_PAD // 128, 128), jnp.float32),
    )(parts)


@functools.partial(
    pl.kernel,
    out_type=jax.ShapeDtypeStruct((NC, E_PAD), jnp.float32),
    mesh=plsc.VectorSubcoreMesh(core_axis_name="c", subcore_axis_name="s"),
    compiler_params=pltpu.CompilerParams(
        use_tc_tiling_on_sc=False, needs_layout_passes=False),
    scratch_types=[
        pltpu.VMEM((NBLK * NSPLIT, BLK // NSPLIT), jnp.int32),
        pltpu.VMEM((NBLK * NSPLIT, BLK // NSPLIT), jnp.int32),
        pltpu.VMEM_SHARED((NRP, DPH), jnp.int32),
        pltpu.VMEM_SHARED((NRP, DPH), jnp.int32),
        pltpu.VMEM((BLK, DPH), jnp.int32),
        pltpu.VMEM((BLK, DPH), jnp.int32),
        pltpu.VMEM((BLK, DPH), jnp.int32),
        pltpu.VMEM((BLK, DPH), jnp.int32),
        pltpu.VMEM((TPW,), jnp.float32),
        pltpu.SemaphoreType.DMA,
        pltpu.SemaphoreType.DMA,
        pltpu.SemaphoreType.DMA,
        pltpu.SemaphoreType.DMA,
    ],
)
def _sc_dot(u_hbm, v_hbm, src_hbm, dst_hbm, out_hbm,
            idx_u, idx_v, u_sp, v_sp, u0, u1, v0, v1, out_all,
            su0, su1, sv0, sv1):
    cid = lax.axis_index("c")
    sid = lax.axis_index("s")
    lanes = lax.iota(jnp.int32, L)
    zero = jnp.zeros((L,), jnp.float32)
    ubufs, vbufs = (u0, u1), (v0, v1)
    usems, vsems = (su0, su1), (sv0, sv1)

    rs = pl.ds(sid * RPT, RPT)
    pltpu.sync_copy(u_hbm.at[cid, rs], u_sp.at[rs])
    pltpu.sync_copy(v_hbm.at[cid, rs], v_sp.at[rs])
    pltpu.sync_copy(src_hbm.at[sid], idx_u)
    pltpu.sync_copy(dst_hbm.at[sid], idx_v)
    plsc.subcore_barrier()

    def fire(g, b):
        for s in range(NSPLIT):
            sl = pl.ds(s * (BLK // NSPLIT), BLK // NSPLIT)
            pltpu.async_copy(
                u_sp.at[idx_u.at[g * NSPLIT + s]], ubufs[b].at[sl], usems[b])
            pltpu.async_copy(
                v_sp.at[idx_v.at[g * NSPLIT + s]], vbufs[b].at[sl], vsems[b])

    def wait(g, b):
        for s in range(NSPLIT):
            sl = pl.ds(s * (BLK // NSPLIT), BLK // NSPLIT)
            pltpu.make_async_copy(
                u_sp.at[idx_u.at[g * NSPLIT + s]], ubufs[b].at[sl],
                usems[b]).wait()
            pltpu.make_async_copy(
                v_sp.at[idx_v.at[g * NSPLIT + s]], vbufs[b].at[sl],
                vsems[b]).wait()

    fire(0, 0)

    def pair_body(i, carry):
        for b in range(2):
            g = 2 * i + b

            @pl.when(g + 1 < NBLK)
            def _():
                fire(g + 1, 1 - b)

            wait(g, b)
            ur, vr = ubufs[b], vbufs[b]

            def d_body(j, accs):
                res = list(accs)
                for t in range(4):
                    d = j * 4 + t
                    dcol = (jnp.full((L,), d, jnp.int32) + lanes) & (DPH - 1)
                    for grp in range(BLK // L):
                        rows = lanes + grp * L
                        uc = plsc.load_gather(ur, [rows, dcol])
                        vc = plsc.load_gather(vr, [rows, dcol])
                        ua, ub = plsc.unpack(
                            plsc.bitcast(uc, jnp.bfloat16),
                            format=plsc.PackFormat.INTERLEAVED)
                        va, vb = plsc.unpack(
                            plsc.bitcast(vc, jnp.bfloat16),
                            format=plsc.PackFormat.INTERLEAVED)
                        k = grp * 2 + (t & 1)
                        res[k] = res[k] + ua * va + ub * vb
                return tuple(res)

            accs = lax.fori_loop(0, DPH // 4, d_body, (zero,) * (2 * BLK // L))
            for grp in range(BLK // L):
                acc = accs[grp * 2] + accs[grp * 2 + 1]
                out_all[pl.ds(g * BLK + grp * L, L)] = acc
        return carry

    lax.fori_loop(0, NBLK // 2, pair_body, 0)
    pltpu.sync_copy(out_all, out_hbm.at[cid, pl.ds(sid * TPW, TPW)])


def kernel(x_user, x_venue, edge_label_index):
    u_pk = _normalize(x_user)
    v_pk = _normalize(x_venue)
    rpad = ((0, NRP - N_ROWS), (0, 0), (0, 0))
    u3 = jnp.pad(u_pk.reshape(N_ROWS, NC, DPH).transpose(1, 0, 2),
                 ((0, 0),) + rpad[:2])
    v3 = jnp.pad(v_pk.reshape(N_ROWS, NC, DPH).transpose(1, 0, 2),
                 ((0, 0),) + rpad[:2])
    eli = edge_label_index.astype(jnp.int32)
    pad = jnp.zeros((E_PAD - E,), jnp.int32)
    src = jnp.concatenate([eli[0], pad]).reshape(NS, NBLK * NSPLIT,
                                                 BLK // NSPLIT)
    dst = jnp.concatenate([eli[1], pad]).reshape(NS, NBLK * NSPLIT,
                                                 BLK // NSPLIT)
    parts = _sc_dot(u3, v3, src, dst)
    out = _combine(parts.reshape(NC, E_PAD // 128, 128))
    return out.reshape(E_PAD)[:E]


# speedup vs baseline: 1.0795x; 1.0084x over previous
import functools

import jax
import jax.numpy as jnp
from jax import lax
from jax.experimental import pallas as pl
from jax.experimental.pallas import tpu as pltpu
from jax.experimental.pallas import tpu_sc as plsc

N_ROWS = 10000
NRP = 10240
D = 256
DP = 128
E = 160000
EPS = 1e-8

NC, NS, L = 2, 16, 16
DPH = DP // NC
BLK = 64
TPW = 10240
E_PAD = NS * TPW
NBLK = TPW // BLK
HBLK = NBLK // 2
NG = HBLK // 5
RPT = NRP // NS


def _norm_body(x_ref, o_ref):
    x = x_ref[...]
    n = jnp.sqrt(jnp.sum(x * x, axis=1, keepdims=True))
    xn = x / jnp.maximum(n, EPS)
    a = jax.lax.bitcast_convert_type(
        xn[:, :DP].astype(jnp.bfloat16), jnp.uint16).astype(jnp.uint32)
    b = jax.lax.bitcast_convert_type(
        xn[:, DP:].astype(jnp.bfloat16), jnp.uint16).astype(jnp.uint32)
    o_ref[...] = jax.lax.bitcast_convert_type(a | (b << 16), jnp.int32)


def _normalize(x):
    return pl.pallas_call(
        _norm_body,
        grid=(5,),
        in_specs=[pl.BlockSpec((2000, D), lambda i: (i, 0))],
        out_specs=pl.BlockSpec((2000, DP), lambda i: (i, 0)),
        out_shape=jax.ShapeDtypeStruct((N_ROWS, DP), jnp.int32),
    )(x)


def _comb_body(p_ref, o_ref):
    o_ref[...] = p_ref[0] + p_ref[1]


def _combine(parts):
    return pl.pallas_call(
        _comb_body,
        grid=(8,),
        in_specs=[pl.BlockSpec((NC, 160, 128), lambda i: (0, i, 0))],
        out_specs=pl.BlockSpec((160, 128), lambda i: (i, 0)),
        out_shape=jax.ShapeDtypeStruct((E_PAD // 128, 128), jnp.float32),
    )(parts)


@functools.partial(
    pl.kernel,
    out_type=jax.ShapeDtypeStruct((NC, E_PAD), jnp.float32),
    mesh=plsc.VectorSubcoreMesh(core_axis_name="c", subcore_axis_name="s"),
    compiler_params=pltpu.CompilerParams(
        use_tc_tiling_on_sc=False, needs_layout_passes=False),
    scratch_types=[
        pltpu.VMEM((HBLK, BLK), jnp.int32),
        pltpu.VMEM((HBLK, BLK), jnp.int32),
        pltpu.VMEM_SHARED((NRP, DPH), jnp.int32),
        pltpu.VMEM_SHARED((NRP, DPH), jnp.int32),
        pltpu.VMEM((BLK, DPH), jnp.int32),
        pltpu.VMEM((BLK, DPH), jnp.int32),
        pltpu.VMEM((BLK, DPH), jnp.int32),
        pltpu.VMEM((BLK, DPH), jnp.int32),
        pltpu.VMEM((BLK, DPH), jnp.int32),
        pltpu.VMEM((BLK, DPH), jnp.int32),
        pltpu.VMEM((TPW,), jnp.float32),
        pltpu.SemaphoreType.DMA,
        pltpu.SemaphoreType.DMA,
        pltpu.SemaphoreType.DMA,
        pltpu.SemaphoreType.DMA,
        pltpu.SemaphoreType.DMA,
        pltpu.SemaphoreType.DMA,
    ],
)
def _sc_dot(u_hbm, v_hbm, src_hbm, dst_hbm, out_hbm,
            idx_u, idx_v, u_sp, v_sp, u0, u1, v0, v1, uh, vh, out_all,
            su0, su1, sv0, sv1, suh, svh):
    cid = lax.axis_index("c")
    sid = lax.axis_index("s")
    lanes = lax.iota(jnp.int32, L)
    zero = jnp.zeros((L,), jnp.float32)
    ubufs, vbufs = (u0, u1), (v0, v1)
    usems, vsems = (su0, su1), (sv0, sv1)

    rs = pl.ds(sid * RPT, RPT)
    pltpu.sync_copy(u_hbm.at[cid, rs], u_sp.at[rs])
    pltpu.sync_copy(v_hbm.at[cid, rs], v_sp.at[rs])
    plsc.subcore_barrier()

    def fire_s(g, b):
        pltpu.async_copy(u_sp.at[idx_u.at[g]], ubufs[b], usems[b])
        pltpu.async_copy(v_sp.at[idx_v.at[g]], vbufs[b], vsems[b])

    def wait_s(g, b):
        pltpu.make_async_copy(u_sp.at[idx_u.at[g]], ubufs[b], usems[b]).wait()
        pltpu.make_async_copy(v_sp.at[idx_v.at[g]], vbufs[b], vsems[b]).wait()

    def fire_h(g):
        pltpu.async_copy(u_hbm.at[cid].at[idx_u.at[g]], uh, suh)
        pltpu.async_copy(v_hbm.at[cid].at[idx_v.at[g]], vh, svh)

    def wait_h(g):
        pltpu.make_async_copy(u_hbm.at[cid].at[idx_u.at[g]], uh, suh).wait()
        pltpu.make_async_copy(v_hbm.at[cid].at[idx_v.at[g]], vh, svh).wait()

    def compute(gout, ur, vr):
        def d_body(j, accs):
            res = list(accs)
            for t in range(4):
                d = j * 4 + t
                dcol = (jnp.full((L,), d, jnp.int32) + lanes) & (DPH - 1)
                for grp in range(BLK // L):
                    rows = lanes + grp * L
                    uc = plsc.load_gather(ur, [rows, dcol])
                    vc = plsc.load_gather(vr, [rows, dcol])
                    ua, ub = plsc.unpack(
                        plsc.bitcast(uc, jnp.bfloat16),
                        format=plsc.PackFormat.INTERLEAVED)
                    va, vb = plsc.unpack(
                        plsc.bitcast(vc, jnp.bfloat16),
                        format=plsc.PackFormat.INTERLEAVED)
                    k = grp * 2 + (t & 1)
                    res[k] = res[k] + ua * va + ub * vb
            return tuple(res)

        accs = lax.fori_loop(0, DPH // 4, d_body, (zero,) * (2 * BLK // L))
        for grp in range(BLK // L):
            acc = accs[grp * 2] + accs[grp * 2 + 1]
            out_all[pl.ds(gout * BLK + grp * L, L)] = acc

    def run_half(half):
        pltpu.sync_copy(src_hbm.at[sid, half], idx_u)
        pltpu.sync_copy(dst_hbm.at[sid, half], idx_v)
        gout0 = half * HBLK
        fire_s(0, 0)

        def super_body(i, carry):
            gh = 5 * i + 4
            fire_h(gh)
            for j in range(4):
                g = 5 * i + j
                b = j % 2
                g_next = g + 1 if j < 3 else 5 * (i + 1)

                if j < 3:
                    fire_s(g_next, 1 - b)
                else:
                    @pl.when(i + 1 < NG)
                    def _():
                        fire_s(g_next, 1 - b)

                wait_s(g, b)
                compute(gout0 + g, ubufs[b], vbufs[b])
            wait_h(gh)
            compute(gout0 + gh, uh, vh)
            return carry

        lax.fori_loop(0, NG, super_body, 0)

    run_half(0)
    run_half(1)
    pltpu.sync_copy(out_all, out_hbm.at[cid, pl.ds(sid * TPW, TPW)])


def kernel(x_user, x_venue, edge_label_index):
    u_pk = _normalize(x_user)
    v_pk = _normalize(x_venue)
    rpad = ((0, 0), (0, NRP - N_ROWS), (0, 0))
    u3 = jnp.pad(u_pk.reshape(N_ROWS, NC, DPH).transpose(1, 0, 2), rpad)
    v3 = jnp.pad(v_pk.reshape(N_ROWS, NC, DPH).transpose(1, 0, 2), rpad)
    eli = edge_label_index.astype(jnp.int32)
    pad = jnp.zeros((E_PAD - E,), jnp.int32)
    src = jnp.concatenate([eli[0], pad]).reshape(NS, 2, HBLK, BLK)
    dst = jnp.concatenate([eli[1], pad]).reshape(NS, 2, HBLK, BLK)
    parts = _sc_dot(u3, v3, src, dst)
    out = _combine(parts.reshape(NC, E_PAD // 128, 128))
    return out.reshape(E_PAD)[:E]
